# Initial kernel scaffold; baseline (speedup 1.0000x reference)
import jax
import jax.numpy as jnp
from jax.experimental import pallas as pl


def kernel(loglik, means, other_loglik, other_means, exch_inds, resample_inds):
    raise NotImplementedError("write your pallas kernel here")


# speedup vs baseline: 1.0006x; 1.0006x over previous
import functools

import jax
import jax.numpy as jnp
from jax import lax
from jax.experimental import pallas as pl
from jax.experimental.pallas import tpu as pltpu
from jax.experimental.pallas import tpu_sc as plsc

T, B, D = 50, 16384, 16
NE = 8192
NC, NS, L = 2, 16, 16
NW = NC * NS
CHUNK = B // NW
CW = 128
NCH = CHUNK // CW
LISTCAP = CHUNK + L

_mesh = plsc.VectorSubcoreMesh(core_axis_name="c", subcore_axis_name="s")


@functools.partial(
    pl.kernel,
    out_type=(
        jax.ShapeDtypeStruct((B,), jnp.float32),
        jax.ShapeDtypeStruct((T, B, D), jnp.float32),
    ),
    mesh=_mesh,
    compiler_params=pltpu.CompilerParams(
        needs_layout_passes=False, use_tc_tiling_on_sc=False
    ),
    scratch_types=[
        pltpu.VMEM((B,), jnp.int32),
        pltpu.VMEM((NE,), jnp.int32),
        pltpu.VMEM((B,), jnp.float32),
        pltpu.VMEM((B,), jnp.float32),
        pltpu.VMEM((CHUNK,), jnp.int32),
        pltpu.VMEM((CHUNK,), jnp.float32),
        pltpu.VMEM((LISTCAP,), jnp.int32),
        pltpu.VMEM((LISTCAP,), jnp.int32),
        pltpu.VMEM((LISTCAP,), jnp.int32),
        pltpu.VMEM((LISTCAP,), jnp.int32),
        pltpu.VMEM((NCH, CW), jnp.int32),
        pltpu.VMEM((NCH, CW), jnp.int32),
        pltpu.VMEM((NCH, CW), jnp.int32),
        pltpu.VMEM((NCH, CW), jnp.int32),
        pltpu.VMEM((CW, D), jnp.float32),
        pltpu.SemaphoreType.DMA,
        pltpu.SemaphoreType.DMA,
    ],
)
def _exchange_resample(
    ll_hbm, oll_hbm, means_hbm, omeans_hbm, exch_hbm, rs_hbm,
    outll_hbm, outms_hbm,
    mask_v, exch_v, ll_v, oll_v, rs_v, outll_v,
    src0f, pos0f, src1f, pos1f,
    src0c, pos0c, src1c, pos1c,
    gbuf, gsem, ssem,
):
    wid = lax.axis_index("s") * NC + lax.axis_index("c")
    base = wid * CHUNK

    pltpu.sync_copy(exch_hbm, exch_v)
    pltpu.sync_copy(ll_hbm, ll_v)
    pltpu.sync_copy(oll_hbm, oll_v)
    pltpu.sync_copy(rs_hbm.at[pl.ds(base, CHUNK)], rs_v)

    zeros16 = jnp.zeros((L,), jnp.int32)
    ones16 = jnp.ones((L,), jnp.int32)
    neg16 = jnp.full((L,), -1, jnp.int32)

    def _zero_mask(i, _):
        mask_v[pl.ds(i * L, L)] = zeros16
        return 0

    lax.fori_loop(0, B // L, _zero_mask, 0)

    def _mark(i, _):
        idx = exch_v[pl.ds(i * L, L)]
        plsc.store_scatter(mask_v, [idx], ones16)
        return 0

    lax.fori_loop(0, NE // L, _mark, 0)

    def _init_lists(i, _):
        src0f[pl.ds(i * L, L)] = zeros16
        src1f[pl.ds(i * L, L)] = zeros16
        pos0f[pl.ds(i * L, L)] = neg16
        pos1f[pl.ds(i * L, L)] = neg16
        return 0

    lax.fori_loop(0, LISTCAP // L, _init_lists, 0)

    lane = lax.iota(jnp.int32, L)

    def _partition(k, carry):
        c0, c1 = carry
        src = rs_v[pl.ds(k * L, L)]
        sel = plsc.load_gather(mask_v, [src])
        m1 = sel != 0
        m0 = jnp.logical_not(m1)
        pos = base + k * L + lane
        plsc.store_compressed(src0f.at[pl.ds(c0, L)], src, mask=m0)
        plsc.store_compressed(pos0f.at[pl.ds(c0, L)], pos, mask=m0)
        plsc.store_compressed(src1f.at[pl.ds(c1, L)], src, mask=m1)
        plsc.store_compressed(pos1f.at[pl.ds(c1, L)], pos, mask=m1)
        lla = plsc.load_gather(ll_v, [src])
        llb = plsc.load_gather(oll_v, [src])
        outll_v[pl.ds(k * L, L)] = jnp.where(m1, llb, lla)
        c0 = c0 + jnp.sum(m0.astype(jnp.int32))
        c1 = c1 + jnp.sum(m1.astype(jnp.int32))
        return (c0, c1)

    n0, n1 = lax.fori_loop(
        0, CHUNK // L, _partition, (jnp.int32(0), jnp.int32(0))
    )

    pltpu.sync_copy(outll_v, outll_hbm.at[pl.ds(base, CHUNK)])

    for c in range(NCH):
        for j in range(CW // L):
            f = c * CW + j * L
            src0c[c, pl.ds(j * L, L)] = src0f[pl.ds(f, L)]
            pos0c[c, pl.ds(j * L, L)] = pos0f[pl.ds(f, L)]
            src1c[c, pl.ds(j * L, L)] = src1f[pl.ds(f, L)]
            pos1c[c, pl.ds(j * L, L)] = pos1f[pl.ds(f, L)]

    def _t_step(t, _):
        for n, s2d, p2d, tab in (
            (n0, src0c, pos0c, means_hbm),
            (n1, src1c, pos1c, omeans_hbm),
        ):
            for c in range(NCH):
                @pl.when(c * CW < n)
                def _():
                    pltpu.async_copy(
                        tab.at[t].at[s2d.at[c]], gbuf, gsem
                    ).wait()
                    pltpu.async_copy(
                        gbuf,
                        outms_hbm.at[t].at[
                            plsc.Indices(p2d.at[c], ignored_value=-1)
                        ],
                        ssem,
                    ).wait()
        return 0

    lax.fori_loop(0, T, _t_step, 0)


@jax.jit
def kernel(loglik, means, other_loglik, other_means, exch_inds, resample_inds):
    out_ll, out_ms = _exchange_resample(
        loglik, other_loglik, means, other_means, exch_inds, resample_inds
    )
    return out_ll, out_ms
